# baseline (speedup 1.0000x reference)
import jax
import jax.numpy as jnp
from jax import lax
from jax.experimental import pallas as pl
from jax.experimental.pallas import tpu as pltpu
from jax.experimental.pallas import tpu_sc as plsc

N = 10000
E = 320000
NC = 2
NS = 16
CHUNK = 128
CPT = 79
EPT = CPT * CHUNK
E_PAD = NC * NS * EPT
N_ACC = 10016
TRASH = 10008
FP = 128

RPT = 632

_mesh = plsc.VectorSubcoreMesh(core_axis_name="c", subcore_axis_name="s")


def _rows_copy(src_ref, dst_ref, s, nrows_last=520):
    @pl.when(s < NS - 1)
    def _():
        pltpu.sync_copy(src_ref.at[pl.ds(s * RPT, RPT)],
                        dst_ref.at[pl.ds(s * RPT, RPT)])

    @pl.when(s == NS - 1)
    def _():
        pltpu.sync_copy(src_ref.at[pl.ds(15 * RPT, nrows_last)],
                        dst_ref.at[pl.ds(15 * RPT, nrows_last)])


def _deg_body(dst_hbm, zeros_hbm, ones_hbm, out_hbm, dst_v, ones_v, acc_sh):
    c = lax.axis_index("c")
    s = lax.axis_index("s")
    wid = c * NS + s
    _rows_copy(zeros_hbm, acc_sh, s, nrows_last=N_ACC - 15 * RPT)
    pltpu.sync_copy(dst_hbm.at[wid], dst_v)
    pltpu.sync_copy(ones_hbm, ones_v)
    plsc.subcore_barrier()

    def chunk(j, _):
        pltpu.sync_copy(ones_v, acc_sh.at[dst_v.at[j]], add=True)
        return 0

    lax.fori_loop(0, CPT, chunk, 0)
    plsc.subcore_barrier()
    _rows_copy(acc_sh, out_hbm.at[c], s)


def _sc_degree(dst3, zeros, ones):
    return pl.kernel(
        _deg_body,
        out_type=jax.ShapeDtypeStruct((NC, N, FP), jnp.float32),
        mesh=_mesh,
        scratch_types=[
            pltpu.VMEM((CPT, CHUNK), jnp.int32),
            pltpu.VMEM((CHUNK, FP), jnp.float32),
            pltpu.VMEM_SHARED((N_ACC, FP), jnp.float32),
        ],
    )(dst3, zeros, ones)


def _seg_body(g_hbm, src_hbm, dst_hbm, out_hbm, src_v, dst_v, rows_v, acc_sh,
              sem):
    c = lax.axis_index("c")
    s = lax.axis_index("s")
    wid = c * NS + s
    _rows_copy(g_hbm, acc_sh, s)
    pltpu.sync_copy(src_hbm.at[wid], src_v)
    pltpu.sync_copy(dst_hbm.at[wid], dst_v)
    plsc.subcore_barrier()

    def chunk(j, _):
        pltpu.async_copy(g_hbm.at[src_v.at[j]], rows_v, sem).wait()
        pltpu.sync_copy(rows_v, acc_sh.at[dst_v.at[j]], add=True)
        return 0

    lax.fori_loop(0, CPT, chunk, 0)
    plsc.subcore_barrier()
    _rows_copy(acc_sh, out_hbm.at[c], s)


def _sc_segment_sum(g, src3, dst3):
    return pl.kernel(
        _seg_body,
        out_type=jax.ShapeDtypeStruct((NC, N, FP), jnp.float32),
        mesh=_mesh,
        scratch_types=[
            pltpu.VMEM((CPT, CHUNK), jnp.int32),
            pltpu.VMEM((CPT, CHUNK), jnp.int32),
            pltpu.VMEM((CHUNK, FP), jnp.float32),
            pltpu.VMEM_SHARED((N_ACC, FP), jnp.float32),
            pltpu.SemaphoreType.DMA,
        ],
    )(g, src3, dst3)


BM = 2000


def _k_dis_body(deg_ref, o_ref):
    deg = 1.0 + deg_ref[0, :, 0:1] + deg_ref[1, :, 0:1]
    o_ref[...] = lax.rsqrt(deg)


def _k_in_body(x_ref, w_ref, dis_ref, o_ref):
    f = w_ref.shape[1]
    r = dis_ref[...] * jnp.dot(x_ref[...], w_ref[...],
                               preferred_element_type=jnp.float32)
    o_ref[...] = jnp.concatenate(
        [r, jnp.zeros((r.shape[0], FP - f), jnp.float32)], axis=1)


def _k_mid_body(acc_ref, g_ref, w_ref, dis_ref, o_ref):
    fin, f = w_ref.shape
    dis = dis_ref[...]
    h = jnp.maximum(
        dis * (acc_ref[0, :, :fin] + acc_ref[1, :, :fin] - g_ref[:, :fin]),
        0.0)
    r = dis * jnp.dot(h, w_ref[...], preferred_element_type=jnp.float32)
    o_ref[...] = jnp.concatenate(
        [r, jnp.zeros((r.shape[0], FP - f), jnp.float32)], axis=1)


def _k_z_body(acc_ref, g_ref, eps_ref, dis_ref, o_ref):
    m = dis_ref[...] * (acc_ref[0, :, :64] + acc_ref[1, :, :64]
                        - g_ref[:, :64])
    o_ref[...] = m[:, :32] + eps_ref[...] * jnp.exp(m[:, 32:])


def _row_call(body, feat_out, in_specs):
    return pl.pallas_call(
        body,
        grid=(N // BM,),
        in_specs=in_specs,
        out_specs=pl.BlockSpec((BM, feat_out), lambda i: (i, 0)),
        out_shape=jax.ShapeDtypeStruct((N, feat_out), jnp.float32),
        compiler_params=pltpu.CompilerParams(
            dimension_semantics=("arbitrary",)),
    )


_acc_spec = pl.BlockSpec((NC, BM, FP), lambda i: (0, i, 0))
_g_spec = pl.BlockSpec((BM, FP), lambda i: (i, 0))
_dis_spec = pl.BlockSpec((BM, 1), lambda i: (i, 0))


def _w_spec(shape):
    return pl.BlockSpec(shape, lambda i: (0, 0))


ZBM = 80


def _k_zz_body(a_ref, b_ref, o_ref):
    o_ref[...] = lax.dot_general(a_ref[...], b_ref[...],
                                 (((1,), (1,)), ((), ())),
                                 preferred_element_type=jnp.float32)


def _decoder(z):
    return pl.pallas_call(
        _k_zz_body,
        grid=(N // ZBM,),
        in_specs=[
            pl.BlockSpec((ZBM, 32), lambda i: (i, 0)),
            pl.BlockSpec((N, 32), lambda i: (0, 0)),
        ],
        out_specs=pl.BlockSpec((ZBM, N), lambda i: (i, 0)),
        out_shape=jax.ShapeDtypeStruct((N, N), jnp.float32),
        compiler_params=pltpu.CompilerParams(
            dimension_semantics=("parallel",)),
    )(z, z)


def kernel(features, edge_index, W1, W2, W3, W4, W5, eps):
    src = edge_index[0].astype(jnp.int32)
    dst = edge_index[1].astype(jnp.int32)
    pad = E_PAD - E
    src3 = jnp.concatenate([src, jnp.zeros((pad,), jnp.int32)]
                           ).reshape(NC * NS, CPT, CHUNK)
    dst3 = jnp.concatenate([dst, jnp.full((pad,), TRASH, jnp.int32)]
                           ).reshape(NC * NS, CPT, CHUNK)
    zeros = jnp.zeros((N_ACC, FP), jnp.float32)
    ones = jnp.ones((CHUNK, FP), jnp.float32)
    W45 = jnp.concatenate([W4, W5], axis=1)

    deg = _sc_degree(dst3, zeros, ones)

    dis = _row_call(_k_dis_body, 1, [
        pl.BlockSpec((NC, BM, FP), lambda i: (0, i, 0))])(deg)

    g1 = _row_call(_k_in_body, FP, [
        pl.BlockSpec((BM, 128), lambda i: (i, 0)), _w_spec((128, 64)),
        _dis_spec])(features, W1, dis)
    a1 = _sc_segment_sum(g1, src3, dst3)

    g2 = _row_call(_k_mid_body, FP, [_acc_spec, _g_spec, _w_spec((64, 32)),
                                     _dis_spec])(a1, g1, W2, dis)
    a2 = _sc_segment_sum(g2, src3, dst3)

    g3 = _row_call(_k_mid_body, FP, [_acc_spec, _g_spec, _w_spec((32, 32)),
                                     _dis_spec])(a2, g2, W3, dis)
    a3 = _sc_segment_sum(g3, src3, dst3)

    g45 = _row_call(_k_mid_body, FP, [_acc_spec, _g_spec, _w_spec((32, 64)),
                                      _dis_spec])(a3, g3, W45, dis)
    a45 = _sc_segment_sum(g45, src3, dst3)

    z = _row_call(_k_z_body, 32, [_acc_spec, _g_spec,
                                  pl.BlockSpec((BM, 32), lambda i: (i, 0)),
                                  _dis_spec])(a45, g45, eps, dis)

    return jnp.reshape(_decoder(z), (-1,))


# speedup vs baseline: 1.0243x; 1.0243x over previous
import jax
import jax.numpy as jnp
from jax import lax
from jax.experimental import pallas as pl
from jax.experimental.pallas import tpu as pltpu
from jax.experimental.pallas import tpu_sc as plsc

N = 10000
E = 320000
NC = 2
NS = 16
CHUNK = 128
CPT = 79
EPT = CPT * CHUNK
E_PAD = NC * NS * EPT
N_ACC = 10016
TRASH = 10008
FP = 128

RPT = 632

_mesh = plsc.VectorSubcoreMesh(core_axis_name="c", subcore_axis_name="s")


def _rows_copy(src_ref, dst_ref, s, nrows_last=520):
    @pl.when(s < NS - 1)
    def _():
        pltpu.sync_copy(src_ref.at[pl.ds(s * RPT, RPT)],
                        dst_ref.at[pl.ds(s * RPT, RPT)])

    @pl.when(s == NS - 1)
    def _():
        pltpu.sync_copy(src_ref.at[pl.ds(15 * RPT, nrows_last)],
                        dst_ref.at[pl.ds(15 * RPT, nrows_last)])


def _deg_body(dst_hbm, zeros_hbm, ones_hbm, out_hbm, dst_v, ones_v, acc_sh):
    c = lax.axis_index("c")
    s = lax.axis_index("s")
    wid = c * NS + s
    _rows_copy(zeros_hbm, acc_sh, s, nrows_last=N_ACC - 15 * RPT)
    pltpu.sync_copy(dst_hbm.at[wid], dst_v)
    pltpu.sync_copy(ones_hbm, ones_v)
    plsc.subcore_barrier()

    def chunk(j, _):
        pltpu.sync_copy(ones_v, acc_sh.at[dst_v.at[j]], add=True)
        return 0

    lax.fori_loop(0, CPT, chunk, 0)
    plsc.subcore_barrier()
    _rows_copy(acc_sh, out_hbm.at[c], s)


def _sc_degree(dst3, zeros, ones):
    return pl.kernel(
        _deg_body,
        out_type=jax.ShapeDtypeStruct((NC, N, FP), jnp.float32),
        mesh=_mesh,
        scratch_types=[
            pltpu.VMEM((CPT, CHUNK), jnp.int32),
            pltpu.VMEM((CHUNK, FP), jnp.float32),
            pltpu.VMEM_SHARED((N_ACC, FP), jnp.float32),
        ],
    )(dst3, zeros, ones)


def _seg_body(g_hbm, src_hbm, dst_hbm, out_hbm, src_v, dst_v, rows_v, acc_sh,
              sem):
    c = lax.axis_index("c")
    s = lax.axis_index("s")
    wid = c * NS + s
    _rows_copy(g_hbm, acc_sh, s)
    pltpu.sync_copy(src_hbm.at[wid], src_v)
    pltpu.sync_copy(dst_hbm.at[wid], dst_v)
    plsc.subcore_barrier()

    def chunk(j, _):
        pltpu.async_copy(g_hbm.at[src_v.at[j]], rows_v, sem).wait()
        pltpu.sync_copy(rows_v, acc_sh.at[dst_v.at[j]], add=True)
        return 0

    lax.fori_loop(0, CPT, chunk, 0)
    plsc.subcore_barrier()
    _rows_copy(acc_sh, out_hbm.at[c], s)


def _sc_segment_sum(g, src3, dst3):
    return pl.kernel(
        _seg_body,
        out_type=jax.ShapeDtypeStruct((NC, N, FP), jnp.float32),
        mesh=_mesh,
        scratch_types=[
            pltpu.VMEM((CPT, CHUNK), jnp.int32),
            pltpu.VMEM((CPT, CHUNK), jnp.int32),
            pltpu.VMEM((CHUNK, FP), jnp.float32),
            pltpu.VMEM_SHARED((N_ACC, FP), jnp.float32),
            pltpu.SemaphoreType.DMA,
        ],
    )(g, src3, dst3)


BM = 2000


def _k_dis_body(deg_ref, o_ref):
    deg = 1.0 + deg_ref[0, :, 0:1] + deg_ref[1, :, 0:1]
    o_ref[...] = lax.rsqrt(deg)


def _k_in_body(x_ref, w_ref, dis_ref, o_ref):
    f = w_ref.shape[1]
    r = dis_ref[...] * jnp.dot(x_ref[...], w_ref[...],
                               preferred_element_type=jnp.float32)
    o_ref[...] = jnp.concatenate(
        [r, jnp.zeros((r.shape[0], FP - f), jnp.float32)], axis=1)


def _k_mid_body(acc_ref, g_ref, w_ref, dis_ref, o_ref):
    fin, f = w_ref.shape
    dis = dis_ref[...]
    h = jnp.maximum(
        dis * (acc_ref[0, :, :fin] + acc_ref[1, :, :fin] - g_ref[:, :fin]),
        0.0)
    r = dis * jnp.dot(h, w_ref[...], preferred_element_type=jnp.float32)
    o_ref[...] = jnp.concatenate(
        [r, jnp.zeros((r.shape[0], FP - f), jnp.float32)], axis=1)


def _k_z_body(acc_ref, g_ref, eps_ref, dis_ref, o_ref):
    m = dis_ref[...] * (acc_ref[0, :, :64] + acc_ref[1, :, :64]
                        - g_ref[:, :64])
    o_ref[...] = m[:, :32] + eps_ref[...] * jnp.exp(m[:, 32:])


def _row_call(body, feat_out, in_specs):
    return pl.pallas_call(
        body,
        grid=(N // BM,),
        in_specs=in_specs,
        out_specs=pl.BlockSpec((BM, feat_out), lambda i: (i, 0)),
        out_shape=jax.ShapeDtypeStruct((N, feat_out), jnp.float32),
        compiler_params=pltpu.CompilerParams(
            dimension_semantics=("arbitrary",)),
    )


_acc_spec = pl.BlockSpec((NC, BM, FP), lambda i: (0, i, 0))
_g_spec = pl.BlockSpec((BM, FP), lambda i: (i, 0))
_dis_spec = pl.BlockSpec((BM, 1), lambda i: (i, 0))


def _w_spec(shape):
    return pl.BlockSpec(shape, lambda i: (0, 0))


ZBM = 400


def _k_zz_body(a_ref, b_ref, o_ref):
    o_ref[...] = lax.dot_general(a_ref[...], b_ref[...],
                                 (((1,), (1,)), ((), ())),
                                 preferred_element_type=jnp.float32)


def _decoder(z):
    return pl.pallas_call(
        _k_zz_body,
        grid=(N // ZBM,),
        in_specs=[
            pl.BlockSpec((ZBM, 32), lambda i: (i, 0)),
            pl.BlockSpec((N, 32), lambda i: (0, 0)),
        ],
        out_specs=pl.BlockSpec((ZBM, N), lambda i: (i, 0)),
        out_shape=jax.ShapeDtypeStruct((N, N), jnp.float32),
        compiler_params=pltpu.CompilerParams(
            dimension_semantics=("parallel",)),
    )(z, z)


def kernel(features, edge_index, W1, W2, W3, W4, W5, eps):
    src = edge_index[0].astype(jnp.int32)
    dst = edge_index[1].astype(jnp.int32)
    pad = E_PAD - E
    src3 = jnp.concatenate([src, jnp.zeros((pad,), jnp.int32)]
                           ).reshape(NC * NS, CPT, CHUNK)
    dst3 = jnp.concatenate([dst, jnp.full((pad,), TRASH, jnp.int32)]
                           ).reshape(NC * NS, CPT, CHUNK)
    zeros = jnp.zeros((N_ACC, FP), jnp.float32)
    ones = jnp.ones((CHUNK, FP), jnp.float32)
    W45 = jnp.concatenate([W4, W5], axis=1)

    deg = _sc_degree(dst3, zeros, ones)

    dis = _row_call(_k_dis_body, 1, [
        pl.BlockSpec((NC, BM, FP), lambda i: (0, i, 0))])(deg)

    g1 = _row_call(_k_in_body, FP, [
        pl.BlockSpec((BM, 128), lambda i: (i, 0)), _w_spec((128, 64)),
        _dis_spec])(features, W1, dis)
    a1 = _sc_segment_sum(g1, src3, dst3)

    g2 = _row_call(_k_mid_body, FP, [_acc_spec, _g_spec, _w_spec((64, 32)),
                                     _dis_spec])(a1, g1, W2, dis)
    a2 = _sc_segment_sum(g2, src3, dst3)

    g3 = _row_call(_k_mid_body, FP, [_acc_spec, _g_spec, _w_spec((32, 32)),
                                     _dis_spec])(a2, g2, W3, dis)
    a3 = _sc_segment_sum(g3, src3, dst3)

    g45 = _row_call(_k_mid_body, FP, [_acc_spec, _g_spec, _w_spec((32, 64)),
                                      _dis_spec])(a3, g3, W45, dis)
    a45 = _sc_segment_sum(g45, src3, dst3)

    z = _row_call(_k_z_body, 32, [_acc_spec, _g_spec,
                                  pl.BlockSpec((BM, 32), lambda i: (i, 0)),
                                  _dis_spec])(a45, g45, eps, dis)

    return jnp.reshape(_decoder(z), (-1,))
